# Initial kernel scaffold; baseline (speedup 1.0000x reference)
import functools

import numpy as np
import jax
import jax.numpy as jnp
from jax import lax
from jax.experimental import pallas as pl
from jax.experimental.pallas import tpu as pltpu
from jax.experimental.pallas import tpu_sc as plsc

N = 4 * 4096 * 768
NC, NS = 2, 16
NW = NC * NS
PER_W = N // NW
CHUNK = 8192
NCH = PER_W // CHUNK
NB1 = 4096
NB2 = 1024
NSLOT = 4


def _quantile_consts():
    nf = np.float32(N)
    out = []
    for q in (0.99, 1.0 - 0.99):
        t = np.float32(np.float32(q) * (nf - np.float32(1)))
        lo, hi = int(np.floor(t)), int(np.ceil(t))
        hw = float(np.float32(t) - np.float32(np.floor(t)))
        lw = float(np.float32(1) - np.float32(hw))
        out.append((lo, hi, lw, hw))
    return out


(_LO_P, _HI_P, _LW_P, _HW_P), (_LO_N, _HI_N, _LW_N, _HW_N) = _quantile_consts()
_RANKS = (_LO_P, _HI_P, _LO_N, _HI_N)

_mesh = plsc.VectorSubcoreMesh(core_axis_name="c", subcore_axis_name="s")


def _keys16(v):
    vu = plsc.bitcast(v, jnp.uint32)
    return jnp.where(v < 0, ~vu, vu | jnp.uint32(0x80000000))


def _fold_stage_reduce(sid, flat_out_hbm, out_base, hist_v, fold_v, stage_sh,
                       rin_v, red_v, nb):
    lanes = lax.iota(jnp.int32, 16)
    zeros16 = jnp.zeros((16,), jnp.int32)
    sl = nb // NS

    def fold_body(i, _):
        acc = zeros16
        for l in range(16):
            acc = acc + plsc.load_gather(hist_v, [(i * 16 + lanes) * 16 + l])
        fold_v[pl.ds(i * 16, 16)] = acc
        return 0

    lax.fori_loop(0, nb // 16, fold_body, 0)

    pltpu.sync_copy(fold_v, stage_sh.at[sid])
    plsc.subcore_barrier()
    pltpu.sync_copy(stage_sh.at[:, pl.ds(sid * sl, sl)], rin_v)

    def red_body(i, _):
        acc = zeros16
        for r in range(16):
            acc = acc + rin_v[r, pl.ds(i * 16, 16)]
        red_v[pl.ds(i * 16, 16)] = acc
        return 0

    lax.fori_loop(0, sl // 16, red_body, 0)
    pltpu.sync_copy(red_v, flat_out_hbm.at[pl.ds(out_base + sid * sl, sl)])


@functools.partial(
    pl.kernel, mesh=_mesh,
    out_type=jax.ShapeDtypeStruct((NC * NB1,), jnp.int32),
    scratch_types=[
        pltpu.VMEM((CHUNK,), jnp.int32),
        pltpu.VMEM((NB1 * 16,), jnp.int32),
        pltpu.VMEM((NB1,), jnp.int32),
        pltpu.VMEM_SHARED((NS, NB1), jnp.int32),
        pltpu.VMEM((NS, NB1 // NS), jnp.int32),
        pltpu.VMEM((NB1 // NS,), jnp.int32),
    ],
)
def _hist1_kernel(x_hbm, out_hbm, data_v, hist_v, fold_v, stage_sh, rin_v,
                  red_v):
    cid = lax.axis_index("c")
    sid = lax.axis_index("s")
    wid = sid * NC + cid
    base = wid * PER_W
    lanes = lax.iota(jnp.int32, 16)
    zeros16 = jnp.zeros((16,), jnp.int32)
    ones16 = jnp.ones((16,), jnp.int32)

    def zbody(i, _):
        hist_v[pl.ds(i * 16, 16)] = zeros16
        return 0

    lax.fori_loop(0, NB1, zbody, 0)

    def chunk_body(ch, _):
        pltpu.sync_copy(x_hbm.at[pl.ds(base + ch * CHUNK, CHUNK)], data_v)

        def vec_body(i, _):
            v = data_v[pl.ds(i * 16, 16)]
            key = _keys16(v)
            row = plsc.bitcast(key >> jnp.uint32(20), jnp.int32)
            plsc.addupdate_scatter(hist_v, [row * 16 + lanes], ones16)
            return 0

        lax.fori_loop(0, CHUNK // 16, vec_body, 0)
        return 0

    lax.fori_loop(0, NCH, chunk_body, 0)

    _fold_stage_reduce(sid, out_hbm, cid * NB1, hist_v, fold_v, stage_sh,
                       rin_v, red_v, NB1)


def _rank_buckets(ha_v, hb_v, nb, ranks):
    nvec = len(ranks)
    zeros16 = jnp.zeros((16,), jnp.int32)

    def body(i, carry):
        tot, cnts, belows = carry
        v = ha_v[pl.ds(i * 16, 16)] + hb_v[pl.ds(i * 16, 16)]
        cs = lax.cumsum(v, axis=0) + tot
        new_cnts = []
        new_belows = []
        for j in range(nvec):
            le = cs <= jnp.int32(ranks[j])
            new_cnts.append(cnts[j] + plsc.all_reduce_population_count(le))
            new_belows.append(jnp.maximum(belows[j], jnp.where(le, cs, 0)))
        return jnp.max(cs), tuple(new_cnts), tuple(new_belows)

    _, cnts, belows = lax.fori_loop(
        0, nb // 16, body,
        (jnp.int32(0), (zeros16,) * nvec, (zeros16,) * nvec))
    belows = tuple(jnp.full((16,), jnp.max(b), jnp.int32) for b in belows)
    return cnts, belows


@functools.partial(
    pl.kernel, mesh=_mesh,
    out_type=jax.ShapeDtypeStruct((NC * NSLOT * NB2,), jnp.int32),
    scratch_types=[
        pltpu.VMEM((CHUNK,), jnp.int32),
        pltpu.VMEM((NSLOT * NB2 * 16,), jnp.int32),
        pltpu.VMEM((NSLOT * NB2,), jnp.int32),
        pltpu.VMEM_SHARED((NS, NSLOT * NB2), jnp.int32),
        pltpu.VMEM((NS, NSLOT * NB2 // NS), jnp.int32),
        pltpu.VMEM((NSLOT * NB2 // NS,), jnp.int32),
        pltpu.VMEM((NB1,), jnp.int32),
        pltpu.VMEM((NB1,), jnp.int32),
    ],
)
def _hist2_kernel(x_hbm, h1_hbm, out_hbm, data_v, hist_v, fold_v, stage_sh,
                  rin_v, red_v, h1a_v, h1b_v):
    cid = lax.axis_index("c")
    sid = lax.axis_index("s")
    wid = sid * NC + cid
    base = wid * PER_W
    lanes = lax.iota(jnp.int32, 16)
    zeros16 = jnp.zeros((16,), jnp.int32)
    ones16 = jnp.ones((16,), jnp.int32)

    pltpu.sync_copy(h1_hbm.at[pl.ds(0, NB1)], h1a_v)
    pltpu.sync_copy(h1_hbm.at[pl.ds(NB1, NB1)], h1b_v)
    (c0, c1, c2, c3), _ = _rank_buckets(h1a_v, h1b_v, NB1, _RANKS)

    s1 = jnp.where(c1 == c0, NB1 + 1, c1)
    s2 = jnp.where((c2 == c0) | (c2 == c1), NB1 + 2, c2)
    s3 = jnp.where((c3 == c0) | (c3 == c1) | (c3 == c2), NB1 + 3, c3)
    base0 = c0 * NB2
    base1 = s1 * NB2
    base2 = s2 * NB2
    base3 = s3 * NB2

    def zbody(i, _):
        hist_v[pl.ds(i * 16, 16)] = zeros16
        return 0

    lax.fori_loop(0, NSLOT * NB2, zbody, 0)

    def chunk_body(ch, _):
        pltpu.sync_copy(x_hbm.at[pl.ds(base + ch * CHUNK, CHUNK)], data_v)

        def vec_body(i, _):
            v = data_v[pl.ds(i * 16, 16)]
            key = _keys16(v)
            g = plsc.bitcast(key >> jnp.uint32(10), jnp.int32)
            d0 = g - base0
            d1 = g - base1
            d2 = g - base2
            d3 = g - base3
            lim = jnp.uint32(NB2)
            m0 = plsc.bitcast(d0, jnp.uint32) < lim
            m1 = plsc.bitcast(d1, jnp.uint32) < lim
            m2 = plsc.bitcast(d2, jnp.uint32) < lim
            m3 = plsc.bitcast(d3, jnp.uint32) < lim
            row = (jnp.where(m0, d0, 0) + jnp.where(m1, d1 + NB2, 0)
                   + jnp.where(m2, d2 + 2 * NB2, 0)
                   + jnp.where(m3, d3 + 3 * NB2, 0))
            plsc.addupdate_scatter(hist_v, [row * 16 + lanes], ones16,
                                   mask=m0 | m1 | m2 | m3)
            return 0

        lax.fori_loop(0, CHUNK // 16, vec_body, 0)
        return 0

    lax.fori_loop(0, NCH, chunk_body, 0)

    _fold_stage_reduce(sid, out_hbm, cid * NSLOT * NB2, hist_v, fold_v,
                       stage_sh, rin_v, red_v, NSLOT * NB2)


@functools.partial(
    pl.kernel, mesh=_mesh,
    out_type=[jax.ShapeDtypeStruct((16,), jnp.float32),
              jax.ShapeDtypeStruct((16,), jnp.float32)],
    scratch_types=[
        pltpu.VMEM((NB1,), jnp.int32),
        pltpu.VMEM((NB1,), jnp.int32),
        pltpu.VMEM((NSLOT * NB2,), jnp.int32),
        pltpu.VMEM((NSLOT * NB2,), jnp.int32),
        pltpu.VMEM((16,), jnp.float32),
        pltpu.VMEM((16,), jnp.float32),
    ],
)
def _finish_kernel(h1_hbm, h2_hbm, outp_hbm, outn_hbm, h1a_v, h1b_v, h2a_v,
                   h2b_v, op_v, on_v):
    cid = lax.axis_index("c")
    sid = lax.axis_index("s")
    wid = sid * NC + cid
    lanes = lax.iota(jnp.int32, 16)

    @pl.when(wid == 0)
    def _():
        pltpu.sync_copy(h1_hbm.at[pl.ds(0, NB1)], h1a_v)
        pltpu.sync_copy(h1_hbm.at[pl.ds(NB1, NB1)], h1b_v)
        pltpu.sync_copy(h2_hbm.at[pl.ds(0, NSLOT * NB2)], h2a_v)
        pltpu.sync_copy(h2_hbm.at[pl.ds(NSLOT * NB2, NSLOT * NB2)], h2b_v)

        cbs, belows = _rank_buckets(h1a_v, h1b_v, NB1, _RANKS)
        c0, c1, c2, c3 = cbs
        srcs = (
            jnp.zeros((16,), jnp.int32),
            jnp.where(c1 == c0, 0, 1),
            jnp.where(c2 == c0, 0, jnp.where(c2 == c1, 1, 2)),
            jnp.where(c3 == c0, 0,
                      jnp.where(c3 == c1, 1, jnp.where(c3 == c2, 2, 3))),
        )

        vals = []
        for j in range(4):
            rwb = jnp.int32(_RANKS[j]) - belows[j]
            slot_base = srcs[j] * NB2

            def cell_body(i, carry, slot_base=slot_base, rwb=rwb):
                tot, cnt = carry
                idx = slot_base + i * 16 + lanes
                v = (plsc.load_gather(h2a_v, [idx])
                     + plsc.load_gather(h2b_v, [idx]))
                cs = lax.cumsum(v, axis=0) + tot
                cnt = cnt + plsc.all_reduce_population_count(cs <= rwb)
                return jnp.max(cs), cnt

            _, cell = lax.fori_loop(0, NB2 // 16, cell_body,
                                    (jnp.int32(0), jnp.zeros((16,), jnp.int32)))
            keyj = (cbs[j] * (1 << 20)) | (cell * (1 << 10)) | jnp.int32(512)
            ku = plsc.bitcast(keyj, jnp.uint32)
            neg = (ku & jnp.uint32(0x80000000)) != jnp.uint32(0)
            bits = jnp.where(neg, ku ^ jnp.uint32(0x80000000), ~ku)
            vals.append(plsc.bitcast(bits, jnp.float32))

        qp = vals[0] * jnp.float32(_LW_P) + vals[1] * jnp.float32(_HW_P)
        qn = vals[2] * jnp.float32(_LW_N) + vals[3] * jnp.float32(_HW_N)
        op_v[...] = qp
        on_v[...] = -qn
        pltpu.sync_copy(op_v, outp_hbm)
        pltpu.sync_copy(on_v, outn_hbm)


def kernel(x, scale_p, scale_n):
    xi = jax.lax.bitcast_convert_type(x, jnp.int32).reshape(N)
    h1 = _hist1_kernel(xi)
    h2 = _hist2_kernel(xi, h1)
    op16, on16 = _finish_kernel(h1, h2)
    return (x, op16[:1], on16[:1])


# speedup vs baseline: 27.0273x; 27.0273x over previous
import functools

import numpy as np
import jax
import jax.numpy as jnp
from jax import lax
from jax.experimental import pallas as pl
from jax.experimental.pallas import tpu as pltpu
from jax.experimental.pallas import tpu_sc as plsc

N = 4 * 4096 * 768
NC, NS = 2, 16
NW = NC * NS
PER_W = N // NW
CHUNK = 8192
NCH = PER_W // CHUNK
NB1 = 4096
NB2 = 1024
NSLOT = 4


def _quantile_consts():
    nf = np.float32(N)
    out = []
    for q in (0.99, 1.0 - 0.99):
        t = np.float32(np.float32(q) * (nf - np.float32(1)))
        lo, hi = int(np.floor(t)), int(np.ceil(t))
        hw = float(np.float32(t) - np.float32(np.floor(t)))
        lw = float(np.float32(1) - np.float32(hw))
        out.append((lo, hi, lw, hw))
    return out


(_LO_P, _HI_P, _LW_P, _HW_P), (_LO_N, _HI_N, _LW_N, _HW_N) = _quantile_consts()
_RANKS = (_LO_P, _HI_P, _LO_N, _HI_N)

_mesh = plsc.VectorSubcoreMesh(core_axis_name="c", subcore_axis_name="s")
_params = pltpu.CompilerParams(needs_layout_passes=False)


def _keys16(v):
    vu = plsc.bitcast(v, jnp.uint32)
    return jnp.where(v < 0, ~vu, vu | jnp.uint32(0x80000000))


def _fold_stage_reduce(sid, flat_out_hbm, out_base, hist_v, fold_v, stage_sh,
                       rin_v, red_v, nb):
    lanes = lax.iota(jnp.int32, 16)
    zeros16 = jnp.zeros((16,), jnp.int32)
    sl = nb // NS

    def fold_body(i, _):
        acc = zeros16
        for l in range(16):
            acc = acc + plsc.load_gather(hist_v, [(i * 16 + lanes) * 16 + l])
        fold_v[pl.ds(i * 16, 16)] = acc
        return 0

    lax.fori_loop(0, nb // 16, fold_body, 0)

    pltpu.sync_copy(fold_v, stage_sh.at[sid])
    plsc.subcore_barrier()
    pltpu.sync_copy(stage_sh.at[:, pl.ds(sid * sl, sl)], rin_v)

    def red_body(i, _):
        acc = zeros16
        for r in range(16):
            acc = acc + rin_v[r, pl.ds(i * 16, 16)]
        red_v[pl.ds(i * 16, 16)] = acc
        return 0

    lax.fori_loop(0, sl // 16, red_body, 0)
    pltpu.sync_copy(red_v, flat_out_hbm.at[pl.ds(out_base + sid * sl, sl)])


@functools.partial(
    pl.kernel, mesh=_mesh, compiler_params=_params,
    out_type=jax.ShapeDtypeStruct((NC * NB1,), jnp.int32),
    scratch_types=[
        pltpu.VMEM((CHUNK,), jnp.int32),
        pltpu.VMEM((NB1 * 16,), jnp.int32),
        pltpu.VMEM((NB1,), jnp.int32),
        pltpu.VMEM_SHARED((NS, NB1), jnp.int32),
        pltpu.VMEM((NS, NB1 // NS), jnp.int32),
        pltpu.VMEM((NB1 // NS,), jnp.int32),
    ],
)
def _hist1_kernel(x_hbm, out_hbm, data_v, hist_v, fold_v, stage_sh, rin_v,
                  red_v):
    cid = lax.axis_index("c")
    sid = lax.axis_index("s")
    wid = sid * NC + cid
    base = wid * PER_W
    lanes = lax.iota(jnp.int32, 16)
    zeros16 = jnp.zeros((16,), jnp.int32)
    ones16 = jnp.ones((16,), jnp.int32)

    def zbody(i, _):
        hist_v[pl.ds(i * 16, 16)] = zeros16
        return 0

    lax.fori_loop(0, NB1, zbody, 0)

    def chunk_body(ch, _):
        pltpu.sync_copy(x_hbm.at[pl.ds(base + ch * CHUNK, CHUNK)], data_v)

        def vec_body(i, _):
            v = data_v[pl.ds(i * 16, 16)]
            key = _keys16(v)
            row = plsc.bitcast(key >> jnp.uint32(20), jnp.int32)
            plsc.addupdate_scatter(hist_v, [row * 16 + lanes], ones16)
            return 0

        lax.fori_loop(0, CHUNK // 16, vec_body, 0)
        return 0

    lax.fori_loop(0, NCH, chunk_body, 0)

    _fold_stage_reduce(sid, out_hbm, cid * NB1, hist_v, fold_v, stage_sh,
                       rin_v, red_v, NB1)


def _rank_buckets(ha_v, hb_v, nb, ranks):
    nvec = len(ranks)
    zeros16 = jnp.zeros((16,), jnp.int32)

    def body(i, carry):
        tot, cnts, belows = carry
        v = ha_v[pl.ds(i * 16, 16)] + hb_v[pl.ds(i * 16, 16)]
        cs = lax.cumsum(v, axis=0) + tot
        new_cnts = []
        new_belows = []
        for j in range(nvec):
            le = cs <= jnp.int32(ranks[j])
            new_cnts.append(cnts[j] + plsc.all_reduce_population_count(le))
            new_belows.append(jnp.maximum(belows[j], jnp.where(le, cs, 0)))
        return jnp.max(cs), tuple(new_cnts), tuple(new_belows)

    _, cnts, belows = lax.fori_loop(
        0, nb // 16, body,
        (jnp.int32(0), (zeros16,) * nvec, (zeros16,) * nvec))
    belows = tuple(jnp.full((16,), jnp.max(b), jnp.int32) for b in belows)
    return cnts, belows


@functools.partial(
    pl.kernel, mesh=_mesh, compiler_params=_params,
    out_type=jax.ShapeDtypeStruct((NC * NSLOT * NB2,), jnp.int32),
    scratch_types=[
        pltpu.VMEM((CHUNK,), jnp.int32),
        pltpu.VMEM((NSLOT * NB2 * 16,), jnp.int32),
        pltpu.VMEM((NSLOT * NB2,), jnp.int32),
        pltpu.VMEM_SHARED((NS, NSLOT * NB2), jnp.int32),
        pltpu.VMEM((NS, NSLOT * NB2 // NS), jnp.int32),
        pltpu.VMEM((NSLOT * NB2 // NS,), jnp.int32),
        pltpu.VMEM((NB1,), jnp.int32),
        pltpu.VMEM((NB1,), jnp.int32),
    ],
)
def _hist2_kernel(x_hbm, h1_hbm, out_hbm, data_v, hist_v, fold_v, stage_sh,
                  rin_v, red_v, h1a_v, h1b_v):
    cid = lax.axis_index("c")
    sid = lax.axis_index("s")
    wid = sid * NC + cid
    base = wid * PER_W
    lanes = lax.iota(jnp.int32, 16)
    zeros16 = jnp.zeros((16,), jnp.int32)
    ones16 = jnp.ones((16,), jnp.int32)

    pltpu.sync_copy(h1_hbm.at[pl.ds(0, NB1)], h1a_v)
    pltpu.sync_copy(h1_hbm.at[pl.ds(NB1, NB1)], h1b_v)
    (c0, c1, c2, c3), _ = _rank_buckets(h1a_v, h1b_v, NB1, _RANKS)

    s1 = jnp.where(c1 == c0, NB1 + 1, c1)
    s2 = jnp.where((c2 == c0) | (c2 == c1), NB1 + 2, c2)
    s3 = jnp.where((c3 == c0) | (c3 == c1) | (c3 == c2), NB1 + 3, c3)
    base0 = c0 * NB2
    base1 = s1 * NB2
    base2 = s2 * NB2
    base3 = s3 * NB2

    def zbody(i, _):
        hist_v[pl.ds(i * 16, 16)] = zeros16
        return 0

    lax.fori_loop(0, NSLOT * NB2, zbody, 0)

    def chunk_body(ch, _):
        pltpu.sync_copy(x_hbm.at[pl.ds(base + ch * CHUNK, CHUNK)], data_v)

        def vec_body(i, _):
            v = data_v[pl.ds(i * 16, 16)]
            key = _keys16(v)
            g = plsc.bitcast(key >> jnp.uint32(10), jnp.int32)
            d0 = g - base0
            d1 = g - base1
            d2 = g - base2
            d3 = g - base3
            lim = jnp.uint32(NB2)
            m0 = plsc.bitcast(d0, jnp.uint32) < lim
            m1 = plsc.bitcast(d1, jnp.uint32) < lim
            m2 = plsc.bitcast(d2, jnp.uint32) < lim
            m3 = plsc.bitcast(d3, jnp.uint32) < lim
            row = (jnp.where(m0, d0, 0) + jnp.where(m1, d1 + NB2, 0)
                   + jnp.where(m2, d2 + 2 * NB2, 0)
                   + jnp.where(m3, d3 + 3 * NB2, 0))
            plsc.addupdate_scatter(hist_v, [row * 16 + lanes], ones16,
                                   mask=m0 | m1 | m2 | m3)
            return 0

        lax.fori_loop(0, CHUNK // 16, vec_body, 0)
        return 0

    lax.fori_loop(0, NCH, chunk_body, 0)

    _fold_stage_reduce(sid, out_hbm, cid * NSLOT * NB2, hist_v, fold_v,
                       stage_sh, rin_v, red_v, NSLOT * NB2)


@functools.partial(
    pl.kernel, mesh=_mesh, compiler_params=_params,
    out_type=[jax.ShapeDtypeStruct((16,), jnp.float32),
              jax.ShapeDtypeStruct((16,), jnp.float32)],
    scratch_types=[
        pltpu.VMEM((NB1,), jnp.int32),
        pltpu.VMEM((NB1,), jnp.int32),
        pltpu.VMEM((NSLOT * NB2,), jnp.int32),
        pltpu.VMEM((NSLOT * NB2,), jnp.int32),
        pltpu.VMEM((16,), jnp.float32),
        pltpu.VMEM((16,), jnp.float32),
    ],
)
def _finish_kernel(h1_hbm, h2_hbm, outp_hbm, outn_hbm, h1a_v, h1b_v, h2a_v,
                   h2b_v, op_v, on_v):
    cid = lax.axis_index("c")
    sid = lax.axis_index("s")
    wid = sid * NC + cid
    lanes = lax.iota(jnp.int32, 16)

    @pl.when(wid == 0)
    def _():
        pltpu.sync_copy(h1_hbm.at[pl.ds(0, NB1)], h1a_v)
        pltpu.sync_copy(h1_hbm.at[pl.ds(NB1, NB1)], h1b_v)
        pltpu.sync_copy(h2_hbm.at[pl.ds(0, NSLOT * NB2)], h2a_v)
        pltpu.sync_copy(h2_hbm.at[pl.ds(NSLOT * NB2, NSLOT * NB2)], h2b_v)

        cbs, belows = _rank_buckets(h1a_v, h1b_v, NB1, _RANKS)
        c0, c1, c2, c3 = cbs
        srcs = (
            jnp.zeros((16,), jnp.int32),
            jnp.where(c1 == c0, 0, 1),
            jnp.where(c2 == c0, 0, jnp.where(c2 == c1, 1, 2)),
            jnp.where(c3 == c0, 0,
                      jnp.where(c3 == c1, 1, jnp.where(c3 == c2, 2, 3))),
        )

        vals = []
        for j in range(4):
            rwb = jnp.int32(_RANKS[j]) - belows[j]
            slot_base = srcs[j] * NB2

            def cell_body(i, carry, slot_base=slot_base, rwb=rwb):
                tot, cnt = carry
                idx = slot_base + i * 16 + lanes
                v = (plsc.load_gather(h2a_v, [idx])
                     + plsc.load_gather(h2b_v, [idx]))
                cs = lax.cumsum(v, axis=0) + tot
                cnt = cnt + plsc.all_reduce_population_count(cs <= rwb)
                return jnp.max(cs), cnt

            _, cell = lax.fori_loop(0, NB2 // 16, cell_body,
                                    (jnp.int32(0), jnp.zeros((16,), jnp.int32)))
            keyj = (cbs[j] * (1 << 20)) | (cell * (1 << 10)) | jnp.int32(512)
            ku = plsc.bitcast(keyj, jnp.uint32)
            neg = (ku & jnp.uint32(0x80000000)) != jnp.uint32(0)
            bits = jnp.where(neg, ku ^ jnp.uint32(0x80000000), ~ku)
            vals.append(plsc.bitcast(bits, jnp.float32))

        qp = vals[0] * jnp.float32(_LW_P) + vals[1] * jnp.float32(_HW_P)
        qn = vals[2] * jnp.float32(_LW_N) + vals[3] * jnp.float32(_HW_N)
        op_v[...] = qp
        on_v[...] = -qn
        pltpu.sync_copy(op_v, outp_hbm)
        pltpu.sync_copy(on_v, outn_hbm)


def kernel(x, scale_p, scale_n):
    xi = jax.lax.bitcast_convert_type(x, jnp.int32).reshape(N)
    h1 = _hist1_kernel(xi)
    h2 = _hist2_kernel(xi, h1)
    op16, on16 = _finish_kernel(h1, h2)
    return (x, op16[:1], on16[:1])


# speedup vs baseline: 32.7341x; 1.2111x over previous
import functools

import numpy as np
import jax
import jax.numpy as jnp
from jax import lax
from jax.experimental import pallas as pl
from jax.experimental.pallas import tpu as pltpu
from jax.experimental.pallas import tpu_sc as plsc

N = 4 * 4096 * 768
NC, NS = 2, 16
NW = NC * NS
PER_W = N // NW
CHUNK = 16384
NCH = PER_W // CHUNK
UNROLL = 8
NB1 = 4096
NB2 = 1024
NSLOT = 4


def _quantile_consts():
    nf = np.float32(N)
    out = []
    for q in (0.99, 1.0 - 0.99):
        t = np.float32(np.float32(q) * (nf - np.float32(1)))
        lo, hi = int(np.floor(t)), int(np.ceil(t))
        hw = float(np.float32(t) - np.float32(np.floor(t)))
        lw = float(np.float32(1) - np.float32(hw))
        out.append((lo, hi, lw, hw))
    return out


(_LO_P, _HI_P, _LW_P, _HW_P), (_LO_N, _HI_N, _LW_N, _HW_N) = _quantile_consts()
_RANKS = (_LO_P, _HI_P, _LO_N, _HI_N)

_mesh = plsc.VectorSubcoreMesh(core_axis_name="c", subcore_axis_name="s")
_params = pltpu.CompilerParams(needs_layout_passes=False)


def _keys16(v):
    vu = plsc.bitcast(v, jnp.uint32)
    return jnp.where(v < 0, ~vu, vu | jnp.uint32(0x80000000))


def _scan_data(x_hbm, d0, d1, semA, semB, base, vec_fn):
    def process(buf):
        def vec_body(i, _):
            for u in range(UNROLL):
                vec_fn(buf[pl.ds((i * UNROLL + u) * 16, 16)])
            return 0

        lax.fori_loop(0, CHUNK // (16 * UNROLL), vec_body, 0)

    pltpu.async_copy(x_hbm.at[pl.ds(base, CHUNK)], d0, semA)
    pltpu.async_copy(x_hbm.at[pl.ds(base + CHUNK, CHUNK)], d1, semB)

    def chunk_body(g, _):
        ch0 = g * 2

        pltpu.make_async_copy(x_hbm.at[pl.ds(0, CHUNK)], d0, semA).wait()
        process(d0)

        @pl.when(ch0 + 2 < NCH)
        def _():
            pltpu.async_copy(
                x_hbm.at[pl.ds(base + (ch0 + 2) * CHUNK, CHUNK)], d0, semA)

        pltpu.make_async_copy(x_hbm.at[pl.ds(0, CHUNK)], d1, semB).wait()
        process(d1)

        @pl.when(ch0 + 3 < NCH)
        def _():
            pltpu.async_copy(
                x_hbm.at[pl.ds(base + (ch0 + 3) * CHUNK, CHUNK)], d1, semB)

        return 0

    lax.fori_loop(0, NCH // 2, chunk_body, 0)


def _zero(ref, nwords):
    zeros16 = jnp.zeros((16,), jnp.int32)

    def zbody(i, _):
        for u in range(UNROLL):
            ref[pl.ds((i * UNROLL + u) * 16, 16)] = zeros16
        return 0

    lax.fori_loop(0, nwords // (16 * UNROLL), zbody, 0)


def _fold_stage_reduce(sid, flat_out_hbm, out_base, hist_v, fold_v, stage_sh,
                       rin_v, red_v, nb):
    lanes = lax.iota(jnp.int32, 16)
    zeros16 = jnp.zeros((16,), jnp.int32)
    sl = nb // NS

    def fold_body(i, _):
        acc = zeros16
        for l in range(16):
            acc = acc + plsc.load_gather(hist_v, [(i * 16 + lanes) * 16 + l])
        fold_v[pl.ds(i * 16, 16)] = acc
        return 0

    lax.fori_loop(0, nb // 16, fold_body, 0)

    pltpu.sync_copy(fold_v, stage_sh.at[sid])
    plsc.subcore_barrier()
    pltpu.sync_copy(stage_sh.at[:, pl.ds(sid * sl, sl)], rin_v)

    def red_body(i, _):
        acc = zeros16
        for r in range(16):
            acc = acc + rin_v[r, pl.ds(i * 16, 16)]
        red_v[pl.ds(i * 16, 16)] = acc
        return 0

    lax.fori_loop(0, sl // 16, red_body, 0)
    pltpu.sync_copy(red_v, flat_out_hbm.at[pl.ds(out_base + sid * sl, sl)])


@functools.partial(
    pl.kernel, mesh=_mesh, compiler_params=_params,
    out_type=jax.ShapeDtypeStruct((NC * NB1,), jnp.int32),
    scratch_types=[
        pltpu.VMEM((CHUNK,), jnp.int32),
        pltpu.VMEM((CHUNK,), jnp.int32),
        pltpu.VMEM((NB1 * 16,), jnp.int32),
        pltpu.VMEM((NB1,), jnp.int32),
        pltpu.VMEM_SHARED((NS, NB1), jnp.int32),
        pltpu.VMEM((NS, NB1 // NS), jnp.int32),
        pltpu.VMEM((NB1 // NS,), jnp.int32),
        pltpu.SemaphoreType.DMA,
        pltpu.SemaphoreType.DMA,
    ],
)
def _hist1_kernel(x_hbm, out_hbm, d0, d1, hist_v, fold_v, stage_sh, rin_v,
                  red_v, semA, semB):
    cid = lax.axis_index("c")
    sid = lax.axis_index("s")
    wid = sid * NC + cid
    lanes = lax.iota(jnp.int32, 16)
    ones16 = jnp.ones((16,), jnp.int32)

    _zero(hist_v, NB1 * 16)

    def vec_fn(v):
        key = _keys16(v)
        row = plsc.bitcast(key >> jnp.uint32(20), jnp.int32)
        plsc.addupdate_scatter(hist_v, [row * 16 + lanes], ones16)

    _scan_data(x_hbm, d0, d1, semA, semB, wid * PER_W, vec_fn)

    _fold_stage_reduce(sid, out_hbm, cid * NB1, hist_v, fold_v, stage_sh,
                       rin_v, red_v, NB1)


def _rank_buckets(ha_v, hb_v, nb, ranks):
    nvec = len(ranks)
    zeros16 = jnp.zeros((16,), jnp.int32)

    def body(i, carry):
        tot, cnts, belows = carry
        v = ha_v[pl.ds(i * 16, 16)] + hb_v[pl.ds(i * 16, 16)]
        cs = lax.cumsum(v, axis=0) + tot
        new_cnts = []
        new_belows = []
        for j in range(nvec):
            le = cs <= jnp.int32(ranks[j])
            new_cnts.append(cnts[j] + plsc.all_reduce_population_count(le))
            new_belows.append(jnp.maximum(belows[j], jnp.where(le, cs, 0)))
        return jnp.max(cs), tuple(new_cnts), tuple(new_belows)

    _, cnts, belows = lax.fori_loop(
        0, nb // 16, body,
        (jnp.int32(0), (zeros16,) * nvec, (zeros16,) * nvec))
    belows = tuple(jnp.full((16,), jnp.max(b), jnp.int32) for b in belows)
    return cnts, belows


@functools.partial(
    pl.kernel, mesh=_mesh, compiler_params=_params,
    out_type=jax.ShapeDtypeStruct((NC * NSLOT * NB2,), jnp.int32),
    scratch_types=[
        pltpu.VMEM((CHUNK,), jnp.int32),
        pltpu.VMEM((CHUNK,), jnp.int32),
        pltpu.VMEM((NSLOT * NB2 * 16,), jnp.int32),
        pltpu.VMEM((NSLOT * NB2,), jnp.int32),
        pltpu.VMEM_SHARED((NS, NSLOT * NB2), jnp.int32),
        pltpu.VMEM((NS, NSLOT * NB2 // NS), jnp.int32),
        pltpu.VMEM((NSLOT * NB2 // NS,), jnp.int32),
        pltpu.VMEM((NB1,), jnp.int32),
        pltpu.VMEM((NB1,), jnp.int32),
        pltpu.SemaphoreType.DMA,
        pltpu.SemaphoreType.DMA,
    ],
)
def _hist2_kernel(x_hbm, h1_hbm, out_hbm, d0, d1, hist_v, fold_v, stage_sh,
                  rin_v, red_v, h1a_v, h1b_v, semA, semB):
    cid = lax.axis_index("c")
    sid = lax.axis_index("s")
    wid = sid * NC + cid
    lanes = lax.iota(jnp.int32, 16)
    ones16 = jnp.ones((16,), jnp.int32)

    pltpu.sync_copy(h1_hbm.at[pl.ds(0, NB1)], h1a_v)
    pltpu.sync_copy(h1_hbm.at[pl.ds(NB1, NB1)], h1b_v)
    (c0, c1, c2, c3), _ = _rank_buckets(h1a_v, h1b_v, NB1, _RANKS)

    s1 = jnp.where(c1 == c0, NB1 + 1, c1)
    s2 = jnp.where((c2 == c0) | (c2 == c1), NB1 + 2, c2)
    s3 = jnp.where((c3 == c0) | (c3 == c1) | (c3 == c2), NB1 + 3, c3)
    base0 = c0 * NB2
    base1 = s1 * NB2
    base2 = s2 * NB2
    base3 = s3 * NB2

    _zero(hist_v, NSLOT * NB2 * 16)

    lim = jnp.uint32(NB2)

    def vec_fn(v):
        key = _keys16(v)
        g = plsc.bitcast(key >> jnp.uint32(10), jnp.int32)
        d0v = g - base0
        d1v = g - base1
        d2v = g - base2
        d3v = g - base3
        m0 = plsc.bitcast(d0v, jnp.uint32) < lim
        m1 = plsc.bitcast(d1v, jnp.uint32) < lim
        m2 = plsc.bitcast(d2v, jnp.uint32) < lim
        m3 = plsc.bitcast(d3v, jnp.uint32) < lim
        row = (jnp.where(m0, d0v, 0) + jnp.where(m1, d1v + NB2, 0)
               + jnp.where(m2, d2v + 2 * NB2, 0)
               + jnp.where(m3, d3v + 3 * NB2, 0))
        plsc.addupdate_scatter(hist_v, [row * 16 + lanes], ones16,
                               mask=m0 | m1 | m2 | m3)

    _scan_data(x_hbm, d0, d1, semA, semB, wid * PER_W, vec_fn)

    _fold_stage_reduce(sid, out_hbm, cid * NSLOT * NB2, hist_v, fold_v,
                       stage_sh, rin_v, red_v, NSLOT * NB2)


@functools.partial(
    pl.kernel, mesh=_mesh, compiler_params=_params,
    out_type=[jax.ShapeDtypeStruct((16,), jnp.float32),
              jax.ShapeDtypeStruct((16,), jnp.float32)],
    scratch_types=[
        pltpu.VMEM((NB1,), jnp.int32),
        pltpu.VMEM((NB1,), jnp.int32),
        pltpu.VMEM((NSLOT * NB2,), jnp.int32),
        pltpu.VMEM((NSLOT * NB2,), jnp.int32),
        pltpu.VMEM((16,), jnp.float32),
        pltpu.VMEM((16,), jnp.float32),
    ],
)
def _finish_kernel(h1_hbm, h2_hbm, outp_hbm, outn_hbm, h1a_v, h1b_v, h2a_v,
                   h2b_v, op_v, on_v):
    cid = lax.axis_index("c")
    sid = lax.axis_index("s")
    wid = sid * NC + cid
    lanes = lax.iota(jnp.int32, 16)

    @pl.when(wid == 0)
    def _():
        pltpu.sync_copy(h1_hbm.at[pl.ds(0, NB1)], h1a_v)
        pltpu.sync_copy(h1_hbm.at[pl.ds(NB1, NB1)], h1b_v)
        pltpu.sync_copy(h2_hbm.at[pl.ds(0, NSLOT * NB2)], h2a_v)
        pltpu.sync_copy(h2_hbm.at[pl.ds(NSLOT * NB2, NSLOT * NB2)], h2b_v)

        cbs, belows = _rank_buckets(h1a_v, h1b_v, NB1, _RANKS)
        c0, c1, c2, c3 = cbs
        srcs = (
            jnp.zeros((16,), jnp.int32),
            jnp.where(c1 == c0, 0, 1),
            jnp.where(c2 == c0, 0, jnp.where(c2 == c1, 1, 2)),
            jnp.where(c3 == c0, 0,
                      jnp.where(c3 == c1, 1, jnp.where(c3 == c2, 2, 3))),
        )

        vals = []
        for j in range(4):
            rwb = jnp.int32(_RANKS[j]) - belows[j]
            slot_base = srcs[j] * NB2

            def cell_body(i, carry, slot_base=slot_base, rwb=rwb):
                tot, cnt = carry
                idx = slot_base + i * 16 + lanes
                v = (plsc.load_gather(h2a_v, [idx])
                     + plsc.load_gather(h2b_v, [idx]))
                cs = lax.cumsum(v, axis=0) + tot
                cnt = cnt + plsc.all_reduce_population_count(cs <= rwb)
                return jnp.max(cs), cnt

            _, cell = lax.fori_loop(0, NB2 // 16, cell_body,
                                    (jnp.int32(0), jnp.zeros((16,), jnp.int32)))
            keyj = (cbs[j] * (1 << 20)) | (cell * (1 << 10)) | jnp.int32(512)
            ku = plsc.bitcast(keyj, jnp.uint32)
            neg = (ku & jnp.uint32(0x80000000)) != jnp.uint32(0)
            bits = jnp.where(neg, ku ^ jnp.uint32(0x80000000), ~ku)
            vals.append(plsc.bitcast(bits, jnp.float32))

        qp = vals[0] * jnp.float32(_LW_P) + vals[1] * jnp.float32(_HW_P)
        qn = vals[2] * jnp.float32(_LW_N) + vals[3] * jnp.float32(_HW_N)
        op_v[...] = qp
        on_v[...] = -qn
        pltpu.sync_copy(op_v, outp_hbm)
        pltpu.sync_copy(on_v, outn_hbm)


def kernel(x, scale_p, scale_n):
    xi = jax.lax.bitcast_convert_type(x, jnp.int32).reshape(N)
    h1 = _hist1_kernel(xi)
    h2 = _hist2_kernel(xi, h1)
    op16, on16 = _finish_kernel(h1, h2)
    return (x, op16[:1], on16[:1])


# speedup vs baseline: 75.3024x; 2.3004x over previous
import functools

import numpy as np
import jax
import jax.numpy as jnp
from jax import lax
from jax.experimental import pallas as pl
from jax.experimental.pallas import tpu as pltpu
from jax.experimental.pallas import tpu_sc as plsc

N = 4 * 4096 * 768
NC, NS = 2, 16
NW = NC * NS
PER_W = N // NW
CHUNK = 16384
NCH = PER_W // CHUNK
UNROLL = 8
NB1 = 4096
NB2 = 1024
NSLOT = 2
INT_MIN = np.int32(-2**31)
INT_MAX = np.int32(2**31 - 1)


def _quantile_consts():
    nf = np.float32(N)
    out = []
    for q in (0.99, 1.0 - 0.99):
        t = np.float32(np.float32(q) * (nf - np.float32(1)))
        lo, hi = int(np.floor(t)), int(np.ceil(t))
        hw = float(np.float32(t) - np.float32(np.floor(t)))
        lw = float(np.float32(1) - np.float32(hw))
        out.append((lo, hi, lw, hw))
    return out


(_LO_P, _HI_P, _LW_P, _HW_P), (_LO_N, _HI_N, _LW_N, _HW_N) = _quantile_consts()
_RANKS = (_LO_P, _LO_N)

_mesh = plsc.VectorSubcoreMesh(core_axis_name="c", subcore_axis_name="s")
_params = pltpu.CompilerParams(needs_layout_passes=False)


def _keys16(v):
    return v ^ ((v >> jnp.int32(31)) | INT_MIN)


def _scan_data(x_hbm, d0, d1, semA, semB, base, vec_fn, hist_v, ones16):
    def process(buf):
        def vec_body(i, _):
            idxs = []
            for u in range(UNROLL):
                idxs.append(vec_fn(buf[pl.ds((i * UNROLL + u) * 16, 16)]))
            for idx, mask in idxs:
                plsc.addupdate_scatter(hist_v, [idx], ones16, mask=mask)
            return 0

        lax.fori_loop(0, CHUNK // (16 * UNROLL), vec_body, 0)

    pltpu.async_copy(x_hbm.at[pl.ds(base, CHUNK)], d0, semA)
    pltpu.async_copy(x_hbm.at[pl.ds(base + CHUNK, CHUNK)], d1, semB)

    def chunk_body(g, _):
        ch0 = g * 2

        pltpu.make_async_copy(x_hbm.at[pl.ds(0, CHUNK)], d0, semA).wait()
        process(d0)

        @pl.when(ch0 + 2 < NCH)
        def _():
            pltpu.async_copy(
                x_hbm.at[pl.ds(base + (ch0 + 2) * CHUNK, CHUNK)], d0, semA)

        pltpu.make_async_copy(x_hbm.at[pl.ds(0, CHUNK)], d1, semB).wait()
        process(d1)

        @pl.when(ch0 + 3 < NCH)
        def _():
            pltpu.async_copy(
                x_hbm.at[pl.ds(base + (ch0 + 3) * CHUNK, CHUNK)], d1, semB)

        return 0

    lax.fori_loop(0, NCH // 2, chunk_body, 0)


def _zero(ref, nwords):
    zeros16 = jnp.zeros((16,), jnp.int32)

    def zbody(i, _):
        for u in range(UNROLL):
            ref[pl.ds((i * UNROLL + u) * 16, 16)] = zeros16
        return 0

    lax.fori_loop(0, nwords // (16 * UNROLL), zbody, 0)


def _fold_stage_reduce(sid, flat_out_hbm, out_base, hist_v, fold_v, stage_sh,
                       rin_v, red_v, nb):
    lanes = lax.iota(jnp.int32, 16)
    zeros16 = jnp.zeros((16,), jnp.int32)
    sl = nb // NS

    def fold_body(i, _):
        acc = zeros16
        for l in range(16):
            acc = acc + plsc.load_gather(hist_v, [(i * 16 + lanes) * 16 + l])
        fold_v[pl.ds(i * 16, 16)] = acc
        return 0

    lax.fori_loop(0, nb // 16, fold_body, 0)

    pltpu.sync_copy(fold_v, stage_sh.at[sid])
    plsc.subcore_barrier()
    pltpu.sync_copy(stage_sh.at[:, pl.ds(sid * sl, sl)], rin_v)

    def red_body(i, _):
        acc = zeros16
        for r in range(16):
            acc = acc + rin_v[r, pl.ds(i * 16, 16)]
        red_v[pl.ds(i * 16, 16)] = acc
        return 0

    lax.fori_loop(0, sl // 16, red_body, 0)
    pltpu.sync_copy(red_v, flat_out_hbm.at[pl.ds(out_base + sid * sl, sl)])


@functools.partial(
    pl.kernel, mesh=_mesh, compiler_params=_params,
    out_type=jax.ShapeDtypeStruct((NC * NB1,), jnp.int32),
    scratch_types=[
        pltpu.VMEM((CHUNK,), jnp.int32),
        pltpu.VMEM((CHUNK,), jnp.int32),
        pltpu.VMEM((NB1 * 16,), jnp.int32),
        pltpu.VMEM((NB1,), jnp.int32),
        pltpu.VMEM_SHARED((NS, NB1), jnp.int32),
        pltpu.VMEM((NS, NB1 // NS), jnp.int32),
        pltpu.VMEM((NB1 // NS,), jnp.int32),
        pltpu.SemaphoreType.DMA,
        pltpu.SemaphoreType.DMA,
    ],
)
def _hist1_kernel(x_hbm, out_hbm, d0, d1, hist_v, fold_v, stage_sh, rin_v,
                  red_v, semA, semB):
    cid = lax.axis_index("c")
    sid = lax.axis_index("s")
    wid = sid * NC + cid
    lanes = lax.iota(jnp.int32, 16)
    ones16 = jnp.ones((16,), jnp.int32)

    _zero(hist_v, NB1 * 16)

    def vec_fn(v):
        key = _keys16(v)
        ku = plsc.bitcast(key, jnp.uint32) >> jnp.uint32(16)
        idx = (plsc.bitcast(ku, jnp.int32) & jnp.int32(0xFFF0)) | lanes
        return idx, None

    _scan_data(x_hbm, d0, d1, semA, semB, wid * PER_W, vec_fn, hist_v, ones16)

    _fold_stage_reduce(sid, out_hbm, cid * NB1, hist_v, fold_v, stage_sh,
                       rin_v, red_v, NB1)


def _rank_buckets(ha_v, hb_v, nb, ranks):
    nvec = len(ranks)
    zeros16 = jnp.zeros((16,), jnp.int32)
    maxs16 = jnp.full((16,), INT_MAX, jnp.int32)

    def body(i, carry):
        tot, cnts, belows, tops = carry
        v = ha_v[pl.ds(i * 16, 16)] + hb_v[pl.ds(i * 16, 16)]
        cs = lax.cumsum(v, axis=0) + tot
        new = []
        for j in range(nvec):
            le = cs <= jnp.int32(ranks[j])
            new.append((cnts[j] + plsc.all_reduce_population_count(le),
                        jnp.maximum(belows[j], jnp.where(le, cs, 0)),
                        jnp.minimum(tops[j], jnp.where(le, INT_MAX, cs))))
        return (jnp.max(cs), tuple(n[0] for n in new),
                tuple(n[1] for n in new), tuple(n[2] for n in new))

    _, cnts, belows, tops = lax.fori_loop(
        0, nb // 16, body,
        (jnp.int32(0), (zeros16,) * nvec, (zeros16,) * nvec,
         (maxs16,) * nvec))
    belows = tuple(jnp.full((16,), jnp.max(b), jnp.int32) for b in belows)
    tops = tuple(jnp.full((16,), jnp.min(t), jnp.int32) for t in tops)
    return cnts, belows, tops


@functools.partial(
    pl.kernel, mesh=_mesh, compiler_params=_params,
    out_type=jax.ShapeDtypeStruct((NC * NSLOT * NB2,), jnp.int32),
    scratch_types=[
        pltpu.VMEM((CHUNK,), jnp.int32),
        pltpu.VMEM((CHUNK,), jnp.int32),
        pltpu.VMEM((NSLOT * NB2 * 16,), jnp.int32),
        pltpu.VMEM((NSLOT * NB2,), jnp.int32),
        pltpu.VMEM_SHARED((NS, NSLOT * NB2), jnp.int32),
        pltpu.VMEM((NS, NSLOT * NB2 // NS), jnp.int32),
        pltpu.VMEM((NSLOT * NB2 // NS,), jnp.int32),
        pltpu.VMEM((NB1,), jnp.int32),
        pltpu.VMEM((NB1,), jnp.int32),
        pltpu.SemaphoreType.DMA,
        pltpu.SemaphoreType.DMA,
    ],
)
def _hist2_kernel(x_hbm, h1_hbm, out_hbm, d0, d1, hist_v, fold_v, stage_sh,
                  rin_v, red_v, h1a_v, h1b_v, semA, semB):
    cid = lax.axis_index("c")
    sid = lax.axis_index("s")
    wid = sid * NC + cid
    lanes = lax.iota(jnp.int32, 16)
    ones16 = jnp.ones((16,), jnp.int32)

    pltpu.sync_copy(h1_hbm.at[pl.ds(0, NB1)], h1a_v)
    pltpu.sync_copy(h1_hbm.at[pl.ds(NB1, NB1)], h1b_v)
    (c0, c2), _, _ = _rank_buckets(h1a_v, h1b_v, NB1, _RANKS)

    s2 = jnp.where(c2 == c0, NB1 + 1, c2)
    base0 = c0 * NB2
    base2 = s2 * NB2
    lim = jnp.uint32(NB2)

    _zero(hist_v, NSLOT * NB2 * 16)

    def vec_fn(v):
        key = _keys16(v)
        g = plsc.bitcast(plsc.bitcast(key, jnp.uint32) >> jnp.uint32(10),
                         jnp.int32)
        dv0 = g - base0
        dv2 = g - base2
        m0 = plsc.bitcast(dv0, jnp.uint32) < lim
        m2 = plsc.bitcast(dv2, jnp.uint32) < lim
        row = jnp.where(m2, dv2 + NB2, dv0)
        idx = (row << jnp.int32(4)) | lanes
        return idx, m0 | m2

    _scan_data(x_hbm, d0, d1, semA, semB, wid * PER_W, vec_fn, hist_v, ones16)

    _fold_stage_reduce(sid, out_hbm, cid * NSLOT * NB2, hist_v, fold_v,
                       stage_sh, rin_v, red_v, NSLOT * NB2)


@functools.partial(
    pl.kernel, mesh=_mesh, compiler_params=_params,
    out_type=[jax.ShapeDtypeStruct((16,), jnp.float32),
              jax.ShapeDtypeStruct((16,), jnp.float32)],
    scratch_types=[
        pltpu.VMEM((NB1,), jnp.int32),
        pltpu.VMEM((NB1,), jnp.int32),
        pltpu.VMEM((NSLOT * NB2,), jnp.int32),
        pltpu.VMEM((NSLOT * NB2,), jnp.int32),
        pltpu.VMEM((16,), jnp.float32),
        pltpu.VMEM((16,), jnp.float32),
    ],
)
def _finish_kernel(h1_hbm, h2_hbm, outp_hbm, outn_hbm, h1a_v, h1b_v, h2a_v,
                   h2b_v, op_v, on_v):
    cid = lax.axis_index("c")
    sid = lax.axis_index("s")
    wid = sid * NC + cid
    lanes = lax.iota(jnp.int32, 16)

    @pl.when(wid == 0)
    def _():
        pltpu.sync_copy(h1_hbm.at[pl.ds(0, NB1)], h1a_v)
        pltpu.sync_copy(h1_hbm.at[pl.ds(NB1, NB1)], h1b_v)
        pltpu.sync_copy(h2_hbm.at[pl.ds(0, NSLOT * NB2)], h2a_v)
        pltpu.sync_copy(h2_hbm.at[pl.ds(NSLOT * NB2, NSLOT * NB2)], h2b_v)

        (c0, c2), belows, tops = _rank_buckets(h1a_v, h1b_v, NB1, _RANKS)
        slot2 = jnp.where(c2 == c0, 0, 1)

        vals = []
        for (slot_base, bucket, below, top, rlo, rhi) in (
                (jnp.zeros((16,), jnp.int32), c0, belows[0], tops[0],
                 _LO_P, _HI_P),
                (slot2 * NB2, c2, belows[1], tops[1], _LO_N, _HI_N)):
            cnt = top - below
            rwb_lo = jnp.minimum(jnp.int32(rlo) - below, cnt - 1)
            rwb_hi = jnp.minimum(jnp.int32(rhi) - below, cnt - 1)

            def cell_body(i, carry, slot_base=slot_base, rwb_lo=rwb_lo,
                          rwb_hi=rwb_hi):
                tot, clo, chi = carry
                idx = slot_base + i * 16 + lanes
                v = (plsc.load_gather(h2a_v, [idx])
                     + plsc.load_gather(h2b_v, [idx]))
                cs = lax.cumsum(v, axis=0) + tot
                clo = clo + plsc.all_reduce_population_count(cs <= rwb_lo)
                chi = chi + plsc.all_reduce_population_count(cs <= rwb_hi)
                return jnp.max(cs), clo, chi

            zeros16 = jnp.zeros((16,), jnp.int32)
            _, cell_lo, cell_hi = lax.fori_loop(
                0, NB2 // 16, cell_body, (jnp.int32(0), zeros16, zeros16))
            for cell in (cell_lo, cell_hi):
                keyj = ((bucket * jnp.int32(1 << 20))
                        | (cell * jnp.int32(1 << 10)) | jnp.int32(512))
                ku = plsc.bitcast(keyj, jnp.uint32)
                neg = (ku & jnp.uint32(0x80000000)) != jnp.uint32(0)
                bits = jnp.where(neg, ku ^ jnp.uint32(0x80000000), ~ku)
                vals.append(plsc.bitcast(bits, jnp.float32))

        qp = vals[0] * jnp.float32(_LW_P) + vals[1] * jnp.float32(_HW_P)
        qn = vals[2] * jnp.float32(_LW_N) + vals[3] * jnp.float32(_HW_N)
        op_v[...] = qp
        on_v[...] = -qn
        pltpu.sync_copy(op_v, outp_hbm)
        pltpu.sync_copy(on_v, outn_hbm)


def kernel(x, scale_p, scale_n):
    xi = jax.lax.bitcast_convert_type(x, jnp.int32).reshape(N)
    h1 = _hist1_kernel(xi)
    h2 = _hist2_kernel(xi, h1)
    op16, on16 = _finish_kernel(h1, h2)
    return (x, op16[:1], on16[:1])


# speedup vs baseline: 93.9165x; 1.2472x over previous
import functools

import numpy as np
import jax
import jax.numpy as jnp
from jax import lax
from jax.experimental import pallas as pl
from jax.experimental.pallas import tpu as pltpu
from jax.experimental.pallas import tpu_sc as plsc

N = 4 * 4096 * 768
B, R, C = 4, 4096, 768
NC, NS = 2, 16
NW = NC * NS
ROWS_W = (B * R) // NW
RWS = 32
NCH = ROWS_W // RWS
UNROLL = 8
GRP = C // (16 * UNROLL)
NB1 = 4096
NB2 = 1024
NSLOT = 2
INT_MIN = np.int32(-2**31)
INT_MAX = np.int32(2**31 - 1)


def _quantile_consts():
    nf = np.float32(N)
    out = []
    for q in (0.99, 1.0 - 0.99):
        t = np.float32(np.float32(q) * (nf - np.float32(1)))
        lo, hi = int(np.floor(t)), int(np.ceil(t))
        hw = float(np.float32(t) - np.float32(np.floor(t)))
        lw = float(np.float32(1) - np.float32(hw))
        out.append((lo, hi, lw, hw))
    return out


(_LO_P, _HI_P, _LW_P, _HW_P), (_LO_N, _HI_N, _LW_N, _HW_N) = _quantile_consts()
_RANKS = (_LO_P, _LO_N)

_mesh = plsc.VectorSubcoreMesh(core_axis_name="c", subcore_axis_name="s")
_params = pltpu.CompilerParams(needs_layout_passes=False)
_params_tt = pltpu.CompilerParams(needs_layout_passes=False,
                                  use_tc_tiling_on_sc=True)


def _keys16(v):
    return v ^ ((v >> jnp.int32(31)) | INT_MIN)


def _scan_data(x_hbm, d0, d1, semA, semB, wid, vec_fn, hist_v, ones16):
    wpb = R // ROWS_W
    b = wid // wpb
    r0 = (wid % wpb) * ROWS_W

    def process(buf):
        def row_body(ri, _):
            for gi in range(GRP):
                idxs = [vec_fn(buf[ri, pl.ds((gi * UNROLL + u) * 16, 16)])
                        for u in range(UNROLL)]
                for idx, mask in idxs:
                    plsc.addupdate_scatter(hist_v, [idx], ones16, mask=mask)
            return 0

        lax.fori_loop(0, RWS, row_body, 0)

    pltpu.async_copy(x_hbm.at[b, pl.ds(r0, RWS), :], d0, semA)
    pltpu.async_copy(x_hbm.at[b, pl.ds(r0 + RWS, RWS), :], d1, semB)

    def chunk_body(g, _):
        ch0 = g * 2

        pltpu.make_async_copy(x_hbm.at[0, pl.ds(0, RWS), :], d0, semA).wait()
        process(d0)

        @pl.when(ch0 + 2 < NCH)
        def _():
            pltpu.async_copy(
                x_hbm.at[b, pl.ds(r0 + (ch0 + 2) * RWS, RWS), :], d0, semA)

        pltpu.make_async_copy(x_hbm.at[0, pl.ds(0, RWS), :], d1, semB).wait()
        process(d1)

        @pl.when(ch0 + 3 < NCH)
        def _():
            pltpu.async_copy(
                x_hbm.at[b, pl.ds(r0 + (ch0 + 3) * RWS, RWS), :], d1, semB)

        return 0

    lax.fori_loop(0, NCH // 2, chunk_body, 0)


def _zero(ref, nwords):
    zeros16 = jnp.zeros((16,), jnp.int32)

    def zbody(i, _):
        for u in range(UNROLL):
            ref[pl.ds((i * UNROLL + u) * 16, 16)] = zeros16
        return 0

    lax.fori_loop(0, nwords // (16 * UNROLL), zbody, 0)


def _fold_stage_reduce(sid, flat_out_hbm, out_base, hist_v, fold_v, stage_sh,
                       rin_v, red_v, nb):
    lanes = lax.iota(jnp.int32, 16)
    zeros16 = jnp.zeros((16,), jnp.int32)
    sl = nb // NS

    def fold_body(i, _):
        acc = zeros16
        for l in range(16):
            acc = acc + plsc.load_gather(hist_v, [(i * 16 + lanes) * 16 + l])
        fold_v[pl.ds(i * 16, 16)] = acc
        return 0

    lax.fori_loop(0, nb // 16, fold_body, 0)

    pltpu.sync_copy(fold_v, stage_sh.at[sid])
    plsc.subcore_barrier()
    pltpu.sync_copy(stage_sh.at[:, pl.ds(sid * sl, sl)], rin_v)

    def red_body(i, _):
        acc = zeros16
        for r in range(16):
            acc = acc + rin_v[r, pl.ds(i * 16, 16)]
        red_v[pl.ds(i * 16, 16)] = acc
        return 0

    lax.fori_loop(0, sl // 16, red_body, 0)
    pltpu.sync_copy(red_v, flat_out_hbm.at[pl.ds(out_base + sid * sl, sl)])


@functools.partial(
    pl.kernel, mesh=_mesh, compiler_params=_params_tt,
    out_type=jax.ShapeDtypeStruct((NC * NB1,), jnp.int32),
    scratch_types=[
        pltpu.VMEM((RWS, C), jnp.float32),
        pltpu.VMEM((RWS, C), jnp.float32),
        pltpu.VMEM((NB1 * 16,), jnp.int32),
        pltpu.VMEM((NB1,), jnp.int32),
        pltpu.VMEM_SHARED((NS, NB1), jnp.int32),
        pltpu.VMEM((NS, NB1 // NS), jnp.int32),
        pltpu.VMEM((NB1 // NS,), jnp.int32),
        pltpu.SemaphoreType.DMA,
        pltpu.SemaphoreType.DMA,
    ],
)
def _hist1_kernel(x_hbm, out_hbm, d0, d1, hist_v, fold_v, stage_sh, rin_v,
                  red_v, semA, semB):
    cid = lax.axis_index("c")
    sid = lax.axis_index("s")
    wid = sid * NC + cid
    lanes = lax.iota(jnp.int32, 16)
    ones16 = jnp.ones((16,), jnp.int32)

    _zero(hist_v, NB1 * 16)

    def vec_fn(v):
        key = _keys16(plsc.bitcast(v, jnp.int32))
        ku = plsc.bitcast(key, jnp.uint32) >> jnp.uint32(16)
        idx = (plsc.bitcast(ku, jnp.int32) & jnp.int32(0xFFF0)) | lanes
        return idx, None

    _scan_data(x_hbm, d0, d1, semA, semB, wid, vec_fn, hist_v, ones16)

    _fold_stage_reduce(sid, out_hbm, cid * NB1, hist_v, fold_v, stage_sh,
                       rin_v, red_v, NB1)


def _rank_buckets(ha_v, hb_v, nb, ranks):
    nvec = len(ranks)
    zeros16 = jnp.zeros((16,), jnp.int32)
    maxs16 = jnp.full((16,), INT_MAX, jnp.int32)

    def body(i, carry):
        tot, cnts, belows, tops = carry
        v = ha_v[pl.ds(i * 16, 16)] + hb_v[pl.ds(i * 16, 16)]
        cs = lax.cumsum(v, axis=0) + tot
        new = []
        for j in range(nvec):
            le = cs <= jnp.int32(ranks[j])
            new.append((cnts[j] + plsc.all_reduce_population_count(le),
                        jnp.maximum(belows[j], jnp.where(le, cs, 0)),
                        jnp.minimum(tops[j], jnp.where(le, INT_MAX, cs))))
        return (jnp.max(cs), tuple(n[0] for n in new),
                tuple(n[1] for n in new), tuple(n[2] for n in new))

    _, cnts, belows, tops = lax.fori_loop(
        0, nb // 16, body,
        (jnp.int32(0), (zeros16,) * nvec, (zeros16,) * nvec,
         (maxs16,) * nvec))
    belows = tuple(jnp.full((16,), jnp.max(b), jnp.int32) for b in belows)
    tops = tuple(jnp.full((16,), jnp.min(t), jnp.int32) for t in tops)
    return cnts, belows, tops


@functools.partial(
    pl.kernel, mesh=_mesh, compiler_params=_params_tt,
    out_type=jax.ShapeDtypeStruct((NC * NSLOT * NB2,), jnp.int32),
    scratch_types=[
        pltpu.VMEM((RWS, C), jnp.float32),
        pltpu.VMEM((RWS, C), jnp.float32),
        pltpu.VMEM((NSLOT * NB2 * 16,), jnp.int32),
        pltpu.VMEM((NSLOT * NB2,), jnp.int32),
        pltpu.VMEM_SHARED((NS, NSLOT * NB2), jnp.int32),
        pltpu.VMEM((NS, NSLOT * NB2 // NS), jnp.int32),
        pltpu.VMEM((NSLOT * NB2 // NS,), jnp.int32),
        pltpu.VMEM((NB1,), jnp.int32),
        pltpu.VMEM((NB1,), jnp.int32),
        pltpu.SemaphoreType.DMA,
        pltpu.SemaphoreType.DMA,
    ],
)
def _hist2_kernel(x_hbm, h1_hbm, out_hbm, d0, d1, hist_v, fold_v, stage_sh,
                  rin_v, red_v, h1a_v, h1b_v, semA, semB):
    cid = lax.axis_index("c")
    sid = lax.axis_index("s")
    wid = sid * NC + cid
    lanes = lax.iota(jnp.int32, 16)
    ones16 = jnp.ones((16,), jnp.int32)

    pltpu.sync_copy(h1_hbm.at[pl.ds(0, NB1)], h1a_v)
    pltpu.sync_copy(h1_hbm.at[pl.ds(NB1, NB1)], h1b_v)
    (c0, c2), _, _ = _rank_buckets(h1a_v, h1b_v, NB1, _RANKS)

    s2 = jnp.where(c2 == c0, NB1 + 1, c2)
    base0 = c0 * NB2
    base2 = s2 * NB2
    lim = jnp.uint32(NB2)

    _zero(hist_v, NSLOT * NB2 * 16)

    def vec_fn(v):
        key = _keys16(plsc.bitcast(v, jnp.int32))
        g = plsc.bitcast(plsc.bitcast(key, jnp.uint32) >> jnp.uint32(10),
                         jnp.int32)
        dv0 = g - base0
        dv2 = g - base2
        m0 = plsc.bitcast(dv0, jnp.uint32) < lim
        m2 = plsc.bitcast(dv2, jnp.uint32) < lim
        row = jnp.where(m2, dv2 + NB2, dv0)
        idx = (row << jnp.int32(4)) | lanes
        return idx, m0 | m2

    _scan_data(x_hbm, d0, d1, semA, semB, wid, vec_fn, hist_v, ones16)

    _fold_stage_reduce(sid, out_hbm, cid * NSLOT * NB2, hist_v, fold_v,
                       stage_sh, rin_v, red_v, NSLOT * NB2)


@functools.partial(
    pl.kernel, mesh=_mesh, compiler_params=_params,
    out_type=[jax.ShapeDtypeStruct((16,), jnp.float32),
              jax.ShapeDtypeStruct((16,), jnp.float32)],
    scratch_types=[
        pltpu.VMEM((NB1,), jnp.int32),
        pltpu.VMEM((NB1,), jnp.int32),
        pltpu.VMEM((NSLOT * NB2,), jnp.int32),
        pltpu.VMEM((NSLOT * NB2,), jnp.int32),
        pltpu.VMEM((16,), jnp.float32),
        pltpu.VMEM((16,), jnp.float32),
    ],
)
def _finish_kernel(h1_hbm, h2_hbm, outp_hbm, outn_hbm, h1a_v, h1b_v, h2a_v,
                   h2b_v, op_v, on_v):
    cid = lax.axis_index("c")
    sid = lax.axis_index("s")
    wid = sid * NC + cid
    lanes = lax.iota(jnp.int32, 16)

    @pl.when(wid == 0)
    def _():
        pltpu.sync_copy(h1_hbm.at[pl.ds(0, NB1)], h1a_v)
        pltpu.sync_copy(h1_hbm.at[pl.ds(NB1, NB1)], h1b_v)
        pltpu.sync_copy(h2_hbm.at[pl.ds(0, NSLOT * NB2)], h2a_v)
        pltpu.sync_copy(h2_hbm.at[pl.ds(NSLOT * NB2, NSLOT * NB2)], h2b_v)

        (c0, c2), belows, tops = _rank_buckets(h1a_v, h1b_v, NB1, _RANKS)
        slot2 = jnp.where(c2 == c0, 0, 1)

        vals = []
        for (slot_base, bucket, below, top, rlo, rhi) in (
                (jnp.zeros((16,), jnp.int32), c0, belows[0], tops[0],
                 _LO_P, _HI_P),
                (slot2 * NB2, c2, belows[1], tops[1], _LO_N, _HI_N)):
            cnt = top - below
            rwb_lo = jnp.minimum(jnp.int32(rlo) - below, cnt - 1)
            rwb_hi = jnp.minimum(jnp.int32(rhi) - below, cnt - 1)

            def cell_body(i, carry, slot_base=slot_base, rwb_lo=rwb_lo,
                          rwb_hi=rwb_hi):
                tot, clo, chi = carry
                idx = slot_base + i * 16 + lanes
                v = (plsc.load_gather(h2a_v, [idx])
                     + plsc.load_gather(h2b_v, [idx]))
                cs = lax.cumsum(v, axis=0) + tot
                clo = clo + plsc.all_reduce_population_count(cs <= rwb_lo)
                chi = chi + plsc.all_reduce_population_count(cs <= rwb_hi)
                return jnp.max(cs), clo, chi

            zeros16 = jnp.zeros((16,), jnp.int32)
            _, cell_lo, cell_hi = lax.fori_loop(
                0, NB2 // 16, cell_body, (jnp.int32(0), zeros16, zeros16))
            for cell in (cell_lo, cell_hi):
                keyj = ((bucket * jnp.int32(1 << 20))
                        | (cell * jnp.int32(1 << 10)) | jnp.int32(512))
                ku = plsc.bitcast(keyj, jnp.uint32)
                neg = (ku & jnp.uint32(0x80000000)) != jnp.uint32(0)
                bits = jnp.where(neg, ku ^ jnp.uint32(0x80000000), ~ku)
                vals.append(plsc.bitcast(bits, jnp.float32))

        qp = vals[0] * jnp.float32(_LW_P) + vals[1] * jnp.float32(_HW_P)
        qn = vals[2] * jnp.float32(_LW_N) + vals[3] * jnp.float32(_HW_N)
        op_v[...] = qp
        on_v[...] = -qn
        pltpu.sync_copy(op_v, outp_hbm)
        pltpu.sync_copy(on_v, outn_hbm)


def kernel(x, scale_p, scale_n):
    h1 = _hist1_kernel(x)
    h2 = _hist2_kernel(x, h1)
    op16, on16 = _finish_kernel(h1, h2)
    return (x, op16[:1], on16[:1])


# speedup vs baseline: 95.4655x; 1.0165x over previous
import functools

import numpy as np
import jax
import jax.numpy as jnp
from jax import lax
from jax.experimental import pallas as pl
from jax.experimental.pallas import tpu as pltpu
from jax.experimental.pallas import tpu_sc as plsc

N = 4 * 4096 * 768
B, R, C = 4, 4096, 768
NC, NS = 2, 16
NW = NC * NS
ROWS_W = (B * R) // NW
RWS = 32
NCH = ROWS_W // RWS
UNROLL = 8
GRP = C // (16 * UNROLL)
NB1 = 4096
NB2 = 1024
NSLOT = 2
INT_MIN = np.int32(-2**31)
INT_MAX = np.int32(2**31 - 1)


def _quantile_consts():
    nf = np.float32(N)
    out = []
    for q in (0.99, 1.0 - 0.99):
        t = np.float32(np.float32(q) * (nf - np.float32(1)))
        lo, hi = int(np.floor(t)), int(np.ceil(t))
        hw = float(np.float32(t) - np.float32(np.floor(t)))
        lw = float(np.float32(1) - np.float32(hw))
        out.append((lo, hi, lw, hw))
    return out


(_LO_P, _HI_P, _LW_P, _HW_P), (_LO_N, _HI_N, _LW_N, _HW_N) = _quantile_consts()
_RANKS = (_LO_P, _LO_N)

_mesh = plsc.VectorSubcoreMesh(core_axis_name="c", subcore_axis_name="s")
_params = pltpu.CompilerParams(needs_layout_passes=False)
_params_tt = pltpu.CompilerParams(needs_layout_passes=False,
                                  use_tc_tiling_on_sc=True)


def _keys16(v):
    return v ^ ((v >> jnp.int32(31)) | INT_MIN)


def _scan_data(x_hbm, d0, d1, semA, semB, wid, vec_fn, hist_v, ones16):
    wpb = R // ROWS_W
    b = wid // wpb
    r0 = (wid % wpb) * ROWS_W

    def process(buf):
        def row_body(ri, _):
            for gi in range(GRP):
                idxs = [vec_fn(buf[ri, pl.ds((gi * UNROLL + u) * 16, 16)])
                        for u in range(UNROLL)]
                for idx, mask in idxs:
                    plsc.addupdate_scatter(hist_v, [idx], ones16, mask=mask)
            return 0

        lax.fori_loop(0, RWS, row_body, 0)

    pltpu.async_copy(x_hbm.at[b, pl.ds(r0, RWS), :], d0, semA)
    pltpu.async_copy(x_hbm.at[b, pl.ds(r0 + RWS, RWS), :], d1, semB)

    def chunk_body(g, _):
        ch0 = g * 2

        pltpu.make_async_copy(x_hbm.at[0, pl.ds(0, RWS), :], d0, semA).wait()
        process(d0)

        @pl.when(ch0 + 2 < NCH)
        def _():
            pltpu.async_copy(
                x_hbm.at[b, pl.ds(r0 + (ch0 + 2) * RWS, RWS), :], d0, semA)

        pltpu.make_async_copy(x_hbm.at[0, pl.ds(0, RWS), :], d1, semB).wait()
        process(d1)

        @pl.when(ch0 + 3 < NCH)
        def _():
            pltpu.async_copy(
                x_hbm.at[b, pl.ds(r0 + (ch0 + 3) * RWS, RWS), :], d1, semB)

        return 0

    lax.fori_loop(0, NCH // 2, chunk_body, 0)


def _zero(ref, nwords):
    zeros16 = jnp.zeros((16,), jnp.int32)

    def zbody(i, _):
        for u in range(UNROLL):
            ref[pl.ds((i * UNROLL + u) * 16, 16)] = zeros16
        return 0

    lax.fori_loop(0, nwords // (16 * UNROLL), zbody, 0)


def _fold_stage_reduce(sid, flat_out_hbm, out_base, hist_v, fold_v, stage_sh,
                       rin_v, red_v, nb):
    lanes = lax.iota(jnp.int32, 16)
    zeros16 = jnp.zeros((16,), jnp.int32)
    sl = nb // NS

    def fold_body(i, _):
        acc = zeros16
        for l in range(16):
            acc = acc + hist_v[pl.ds(l * nb + i * 16, 16)]
        fold_v[pl.ds(i * 16, 16)] = acc
        return 0

    lax.fori_loop(0, nb // 16, fold_body, 0)

    pltpu.sync_copy(fold_v, stage_sh.at[sid])
    plsc.subcore_barrier()
    pltpu.sync_copy(stage_sh.at[:, pl.ds(sid * sl, sl)], rin_v)

    def red_body(i, _):
        acc = zeros16
        for r in range(16):
            acc = acc + rin_v[r, pl.ds(i * 16, 16)]
        red_v[pl.ds(i * 16, 16)] = acc
        return 0

    lax.fori_loop(0, sl // 16, red_body, 0)
    pltpu.sync_copy(red_v, flat_out_hbm.at[pl.ds(out_base + sid * sl, sl)])


@functools.partial(
    pl.kernel, mesh=_mesh, compiler_params=_params_tt,
    out_type=jax.ShapeDtypeStruct((NC * NB1,), jnp.int32),
    scratch_types=[
        pltpu.VMEM((RWS, C), jnp.float32),
        pltpu.VMEM((RWS, C), jnp.float32),
        pltpu.VMEM((NB1 * 16,), jnp.int32),
        pltpu.VMEM((NB1,), jnp.int32),
        pltpu.VMEM_SHARED((NS, NB1), jnp.int32),
        pltpu.VMEM((NS, NB1 // NS), jnp.int32),
        pltpu.VMEM((NB1 // NS,), jnp.int32),
        pltpu.SemaphoreType.DMA,
        pltpu.SemaphoreType.DMA,
    ],
)
def _hist1_kernel(x_hbm, out_hbm, d0, d1, hist_v, fold_v, stage_sh, rin_v,
                  red_v, semA, semB):
    cid = lax.axis_index("c")
    sid = lax.axis_index("s")
    wid = sid * NC + cid
    lanes = lax.iota(jnp.int32, 16)
    ones16 = jnp.ones((16,), jnp.int32)

    _zero(hist_v, NB1 * 16)
    lanes_nb = lanes * jnp.int32(NB1)

    def vec_fn(v):
        key = _keys16(plsc.bitcast(v, jnp.int32))
        row = plsc.bitcast(plsc.bitcast(key, jnp.uint32) >> jnp.uint32(20),
                           jnp.int32)
        return lanes_nb + row, None

    _scan_data(x_hbm, d0, d1, semA, semB, wid, vec_fn, hist_v, ones16)

    _fold_stage_reduce(sid, out_hbm, cid * NB1, hist_v, fold_v, stage_sh,
                       rin_v, red_v, NB1)


def _rank_buckets(ha_v, hb_v, nb, ranks):
    nvec = len(ranks)
    zeros16 = jnp.zeros((16,), jnp.int32)
    maxs16 = jnp.full((16,), INT_MAX, jnp.int32)

    def body(i, carry):
        tot, cnts, belows, tops = carry
        v = ha_v[pl.ds(i * 16, 16)] + hb_v[pl.ds(i * 16, 16)]
        cs = lax.cumsum(v, axis=0) + tot
        new = []
        for j in range(nvec):
            le = cs <= jnp.int32(ranks[j])
            new.append((cnts[j] + plsc.all_reduce_population_count(le),
                        jnp.maximum(belows[j], jnp.where(le, cs, 0)),
                        jnp.minimum(tops[j], jnp.where(le, INT_MAX, cs))))
        return (jnp.max(cs), tuple(n[0] for n in new),
                tuple(n[1] for n in new), tuple(n[2] for n in new))

    _, cnts, belows, tops = lax.fori_loop(
        0, nb // 16, body,
        (jnp.int32(0), (zeros16,) * nvec, (zeros16,) * nvec,
         (maxs16,) * nvec))
    belows = tuple(jnp.full((16,), jnp.max(b), jnp.int32) for b in belows)
    tops = tuple(jnp.full((16,), jnp.min(t), jnp.int32) for t in tops)
    return cnts, belows, tops


@functools.partial(
    pl.kernel, mesh=_mesh, compiler_params=_params_tt,
    out_type=jax.ShapeDtypeStruct((NC * NSLOT * NB2,), jnp.int32),
    scratch_types=[
        pltpu.VMEM((RWS, C), jnp.float32),
        pltpu.VMEM((RWS, C), jnp.float32),
        pltpu.VMEM((NSLOT * NB2 * 16,), jnp.int32),
        pltpu.VMEM((NSLOT * NB2,), jnp.int32),
        pltpu.VMEM_SHARED((NS, NSLOT * NB2), jnp.int32),
        pltpu.VMEM((NS, NSLOT * NB2 // NS), jnp.int32),
        pltpu.VMEM((NSLOT * NB2 // NS,), jnp.int32),
        pltpu.VMEM((NB1,), jnp.int32),
        pltpu.VMEM((NB1,), jnp.int32),
        pltpu.SemaphoreType.DMA,
        pltpu.SemaphoreType.DMA,
    ],
)
def _hist2_kernel(x_hbm, h1_hbm, out_hbm, d0, d1, hist_v, fold_v, stage_sh,
                  rin_v, red_v, h1a_v, h1b_v, semA, semB):
    cid = lax.axis_index("c")
    sid = lax.axis_index("s")
    wid = sid * NC + cid
    lanes = lax.iota(jnp.int32, 16)
    ones16 = jnp.ones((16,), jnp.int32)

    pltpu.sync_copy(h1_hbm.at[pl.ds(0, NB1)], h1a_v)
    pltpu.sync_copy(h1_hbm.at[pl.ds(NB1, NB1)], h1b_v)
    (c0, c2), _, _ = _rank_buckets(h1a_v, h1b_v, NB1, _RANKS)

    s2 = jnp.where(c2 == c0, NB1 + 1, c2)
    base0 = c0 * NB2
    base2 = s2 * NB2
    lim = jnp.uint32(NB2)

    _zero(hist_v, NSLOT * NB2 * 16)
    lanes_nb = lanes * jnp.int32(NSLOT * NB2)

    def vec_fn(v):
        key = _keys16(plsc.bitcast(v, jnp.int32))
        g = plsc.bitcast(plsc.bitcast(key, jnp.uint32) >> jnp.uint32(10),
                         jnp.int32)
        dv0 = g - base0
        dv2 = g - base2
        m0 = plsc.bitcast(dv0, jnp.uint32) < lim
        m2 = plsc.bitcast(dv2, jnp.uint32) < lim
        row = jnp.where(m2, dv2 + NB2, dv0)
        return lanes_nb + row, m0 | m2

    _scan_data(x_hbm, d0, d1, semA, semB, wid, vec_fn, hist_v, ones16)

    _fold_stage_reduce(sid, out_hbm, cid * NSLOT * NB2, hist_v, fold_v,
                       stage_sh, rin_v, red_v, NSLOT * NB2)


@functools.partial(
    pl.kernel, mesh=_mesh, compiler_params=_params,
    out_type=[jax.ShapeDtypeStruct((16,), jnp.float32),
              jax.ShapeDtypeStruct((16,), jnp.float32)],
    scratch_types=[
        pltpu.VMEM((NB1,), jnp.int32),
        pltpu.VMEM((NB1,), jnp.int32),
        pltpu.VMEM((NSLOT * NB2,), jnp.int32),
        pltpu.VMEM((NSLOT * NB2,), jnp.int32),
        pltpu.VMEM((16,), jnp.float32),
        pltpu.VMEM((16,), jnp.float32),
    ],
)
def _finish_kernel(h1_hbm, h2_hbm, outp_hbm, outn_hbm, h1a_v, h1b_v, h2a_v,
                   h2b_v, op_v, on_v):
    cid = lax.axis_index("c")
    sid = lax.axis_index("s")
    wid = sid * NC + cid
    lanes = lax.iota(jnp.int32, 16)

    @pl.when(wid == 0)
    def _():
        pltpu.sync_copy(h1_hbm.at[pl.ds(0, NB1)], h1a_v)
        pltpu.sync_copy(h1_hbm.at[pl.ds(NB1, NB1)], h1b_v)
        pltpu.sync_copy(h2_hbm.at[pl.ds(0, NSLOT * NB2)], h2a_v)
        pltpu.sync_copy(h2_hbm.at[pl.ds(NSLOT * NB2, NSLOT * NB2)], h2b_v)

        (c0, c2), belows, tops = _rank_buckets(h1a_v, h1b_v, NB1, _RANKS)
        slot2 = jnp.where(c2 == c0, 0, 1)

        vals = []
        for (slot_base, bucket, below, top, rlo, rhi) in (
                (jnp.zeros((16,), jnp.int32), c0, belows[0], tops[0],
                 _LO_P, _HI_P),
                (slot2 * NB2, c2, belows[1], tops[1], _LO_N, _HI_N)):
            cnt = top - below
            rwb_lo = jnp.minimum(jnp.int32(rlo) - below, cnt - 1)
            rwb_hi = jnp.minimum(jnp.int32(rhi) - below, cnt - 1)

            def cell_body(i, carry, slot_base=slot_base, rwb_lo=rwb_lo,
                          rwb_hi=rwb_hi):
                tot, clo, chi = carry
                idx = slot_base + i * 16 + lanes
                v = (plsc.load_gather(h2a_v, [idx])
                     + plsc.load_gather(h2b_v, [idx]))
                cs = lax.cumsum(v, axis=0) + tot
                clo = clo + plsc.all_reduce_population_count(cs <= rwb_lo)
                chi = chi + plsc.all_reduce_population_count(cs <= rwb_hi)
                return jnp.max(cs), clo, chi

            zeros16 = jnp.zeros((16,), jnp.int32)
            _, cell_lo, cell_hi = lax.fori_loop(
                0, NB2 // 16, cell_body, (jnp.int32(0), zeros16, zeros16))
            for cell in (cell_lo, cell_hi):
                keyj = ((bucket * jnp.int32(1 << 20))
                        | (cell * jnp.int32(1 << 10)) | jnp.int32(512))
                ku = plsc.bitcast(keyj, jnp.uint32)
                neg = (ku & jnp.uint32(0x80000000)) != jnp.uint32(0)
                bits = jnp.where(neg, ku ^ jnp.uint32(0x80000000), ~ku)
                vals.append(plsc.bitcast(bits, jnp.float32))

        qp = vals[0] * jnp.float32(_LW_P) + vals[1] * jnp.float32(_HW_P)
        qn = vals[2] * jnp.float32(_LW_N) + vals[3] * jnp.float32(_HW_N)
        op_v[...] = qp
        on_v[...] = -qn
        pltpu.sync_copy(op_v, outp_hbm)
        pltpu.sync_copy(on_v, outn_hbm)


def kernel(x, scale_p, scale_n):
    x_out = x + jnp.float32(0.0)
    h1 = _hist1_kernel(x)
    h2 = _hist2_kernel(x, h1)
    op16, on16 = _finish_kernel(h1, h2)
    return (x_out, op16[:1], on16[:1])


# speedup vs baseline: 95.8893x; 1.0044x over previous
import functools

import numpy as np
import jax
import jax.numpy as jnp
from jax import lax
from jax.experimental import pallas as pl
from jax.experimental.pallas import tpu as pltpu
from jax.experimental.pallas import tpu_sc as plsc

N = 4 * 4096 * 768
B, R, C = 4, 4096, 768
NC, NS = 2, 16
NW = NC * NS
ROWS_W = (B * R) // NW
RWS = 32
NCH = ROWS_W // RWS
UNROLL = 8
GRP = C // (16 * UNROLL)
NB1 = 4096
NB2 = 1024
NSLOT = 2
INT_MIN = np.int32(-2**31)
INT_MAX = np.int32(2**31 - 1)


def _quantile_consts():
    nf = np.float32(N)
    out = []
    for q in (0.99, 1.0 - 0.99):
        t = np.float32(np.float32(q) * (nf - np.float32(1)))
        lo, hi = int(np.floor(t)), int(np.ceil(t))
        hw = float(np.float32(t) - np.float32(np.floor(t)))
        lw = float(np.float32(1) - np.float32(hw))
        out.append((lo, hi, lw, hw))
    return out


(_LO_P, _HI_P, _LW_P, _HW_P), (_LO_N, _HI_N, _LW_N, _HW_N) = _quantile_consts()
_RANKS = (_LO_P, _LO_N)

_mesh = plsc.VectorSubcoreMesh(core_axis_name="c", subcore_axis_name="s")
_params = pltpu.CompilerParams(needs_layout_passes=False)
_params_tt = pltpu.CompilerParams(needs_layout_passes=False,
                                  use_tc_tiling_on_sc=True)


def _keys16(v):
    return v ^ ((v >> jnp.int32(31)) | INT_MIN)


def _scan_data(x_hbm, d0, d1, semA, semB, wid, vec_fn, hist_v, ones16):
    wpb = R // ROWS_W
    b = wid // wpb
    r0 = (wid % wpb) * ROWS_W

    def process(buf):
        def row_body(ri, _):
            for gi in range(GRP):
                idxs = [vec_fn(buf[ri, pl.ds((gi * UNROLL + u) * 16, 16)])
                        for u in range(UNROLL)]
                for idx, mask in idxs:
                    plsc.addupdate_scatter(hist_v, [idx], ones16, mask=mask)
            return 0

        lax.fori_loop(0, RWS, row_body, 0)

    pltpu.async_copy(x_hbm.at[b, pl.ds(r0, RWS), :], d0, semA)
    pltpu.async_copy(x_hbm.at[b, pl.ds(r0 + RWS, RWS), :], d1, semB)

    def chunk_body(g, _):
        ch0 = g * 2

        pltpu.make_async_copy(x_hbm.at[0, pl.ds(0, RWS), :], d0, semA).wait()
        process(d0)

        @pl.when(ch0 + 2 < NCH)
        def _():
            pltpu.async_copy(
                x_hbm.at[b, pl.ds(r0 + (ch0 + 2) * RWS, RWS), :], d0, semA)

        pltpu.make_async_copy(x_hbm.at[0, pl.ds(0, RWS), :], d1, semB).wait()
        process(d1)

        @pl.when(ch0 + 3 < NCH)
        def _():
            pltpu.async_copy(
                x_hbm.at[b, pl.ds(r0 + (ch0 + 3) * RWS, RWS), :], d1, semB)

        return 0

    lax.fori_loop(0, NCH // 2, chunk_body, 0)


def _zero(ref, nwords):
    zeros16 = jnp.zeros((16,), jnp.int32)

    def zbody(i, _):
        for u in range(UNROLL):
            ref[pl.ds((i * UNROLL + u) * 16, 16)] = zeros16
        return 0

    lax.fori_loop(0, nwords // (16 * UNROLL), zbody, 0)


def _fold_stage_reduce(sid, flat_out_hbm, out_base, hist_v, fold_v, stage_sh,
                       rin_v, red_v, nb):
    lanes = lax.iota(jnp.int32, 16)
    zeros16 = jnp.zeros((16,), jnp.int32)
    sl = nb // NS

    def fold_body(i, _):
        acc = zeros16
        for l in range(16):
            acc = acc + hist_v[pl.ds(l * nb + i * 16, 16)]
        fold_v[pl.ds(i * 16, 16)] = acc
        return 0

    lax.fori_loop(0, nb // 16, fold_body, 0)

    pltpu.sync_copy(fold_v, stage_sh.at[sid])
    plsc.subcore_barrier()
    pltpu.sync_copy(stage_sh.at[:, pl.ds(sid * sl, sl)], rin_v)

    def red_body(i, _):
        acc = zeros16
        for r in range(16):
            acc = acc + rin_v[r, pl.ds(i * 16, 16)]
        red_v[pl.ds(i * 16, 16)] = acc
        return 0

    lax.fori_loop(0, sl // 16, red_body, 0)
    pltpu.sync_copy(red_v, flat_out_hbm.at[pl.ds(out_base + sid * sl, sl)])


@functools.partial(
    pl.kernel, mesh=_mesh, compiler_params=_params_tt,
    out_type=jax.ShapeDtypeStruct((NC * NB1,), jnp.int32),
    scratch_types=[
        pltpu.VMEM((RWS, C), jnp.float32),
        pltpu.VMEM((RWS, C), jnp.float32),
        pltpu.VMEM((NB1 * 16,), jnp.int32),
        pltpu.VMEM((NB1,), jnp.int32),
        pltpu.VMEM_SHARED((NS, NB1), jnp.int32),
        pltpu.VMEM((NS, NB1 // NS), jnp.int32),
        pltpu.VMEM((NB1 // NS,), jnp.int32),
        pltpu.SemaphoreType.DMA,
        pltpu.SemaphoreType.DMA,
    ],
)
def _hist1_kernel(x_hbm, out_hbm, d0, d1, hist_v, fold_v, stage_sh, rin_v,
                  red_v, semA, semB):
    cid = lax.axis_index("c")
    sid = lax.axis_index("s")
    wid = sid * NC + cid
    lanes = lax.iota(jnp.int32, 16)
    ones16 = jnp.ones((16,), jnp.int32)

    _zero(hist_v, NB1 * 16)
    lanes_nb = lanes * jnp.int32(NB1)

    def vec_fn(v):
        key = _keys16(plsc.bitcast(v, jnp.int32))
        row = plsc.bitcast(plsc.bitcast(key, jnp.uint32) >> jnp.uint32(20),
                           jnp.int32)
        return lanes_nb + row, None

    _scan_data(x_hbm, d0, d1, semA, semB, wid, vec_fn, hist_v, ones16)

    _fold_stage_reduce(sid, out_hbm, cid * NB1, hist_v, fold_v, stage_sh,
                       rin_v, red_v, NB1)


def _rank_buckets(ha_v, hb_v, nb, ranks):
    nvec = len(ranks)
    zeros16 = jnp.zeros((16,), jnp.int32)
    maxs16 = jnp.full((16,), INT_MAX, jnp.int32)

    def body(i, carry):
        tot, cnts, belows, tops = carry
        v = ha_v[pl.ds(i * 16, 16)] + hb_v[pl.ds(i * 16, 16)]
        cs = lax.cumsum(v, axis=0) + tot
        new = []
        for j in range(nvec):
            le = cs <= jnp.int32(ranks[j])
            new.append((cnts[j] + plsc.all_reduce_population_count(le),
                        jnp.maximum(belows[j], jnp.where(le, cs, 0)),
                        jnp.minimum(tops[j], jnp.where(le, INT_MAX, cs))))
        return (jnp.max(cs), tuple(n[0] for n in new),
                tuple(n[1] for n in new), tuple(n[2] for n in new))

    _, cnts, belows, tops = lax.fori_loop(
        0, nb // 16, body,
        (jnp.int32(0), (zeros16,) * nvec, (zeros16,) * nvec,
         (maxs16,) * nvec))
    belows = tuple(jnp.full((16,), jnp.max(b), jnp.int32) for b in belows)
    tops = tuple(jnp.full((16,), jnp.min(t), jnp.int32) for t in tops)
    return cnts, belows, tops


@functools.partial(
    pl.kernel, mesh=_mesh, compiler_params=_params_tt,
    out_type=jax.ShapeDtypeStruct((NC * NSLOT * NB2,), jnp.int32),
    scratch_types=[
        pltpu.VMEM((RWS, C), jnp.float32),
        pltpu.VMEM((RWS, C), jnp.float32),
        pltpu.VMEM((NSLOT * NB2 * 16,), jnp.int32),
        pltpu.VMEM((NSLOT * NB2,), jnp.int32),
        pltpu.VMEM_SHARED((NS, NSLOT * NB2), jnp.int32),
        pltpu.VMEM((NS, NSLOT * NB2 // NS), jnp.int32),
        pltpu.VMEM((NSLOT * NB2 // NS,), jnp.int32),
        pltpu.VMEM((NB1,), jnp.int32),
        pltpu.VMEM((NB1,), jnp.int32),
        pltpu.VMEM_SHARED((32,), jnp.int32),
        pltpu.SemaphoreType.DMA,
        pltpu.SemaphoreType.DMA,
    ],
)
def _hist2_kernel(x_hbm, h1_hbm, out_hbm, d0, d1, hist_v, fold_v, stage_sh,
                  rin_v, red_v, h1a_v, h1b_v, bcast_sh, semA, semB):
    cid = lax.axis_index("c")
    sid = lax.axis_index("s")
    wid = sid * NC + cid
    lanes = lax.iota(jnp.int32, 16)
    ones16 = jnp.ones((16,), jnp.int32)

    _zero(hist_v, NSLOT * NB2 * 16)

    @pl.when(sid == 0)
    def _():
        pltpu.sync_copy(h1_hbm.at[pl.ds(0, NB1)], h1a_v)
        pltpu.sync_copy(h1_hbm.at[pl.ds(NB1, NB1)], h1b_v)
        (c0, c2), _, _ = _rank_buckets(h1a_v, h1b_v, NB1, _RANKS)
        h1a_v[pl.ds(0, 16)] = c0
        h1a_v[pl.ds(16, 16)] = c2
        pltpu.sync_copy(h1a_v.at[pl.ds(0, 32)], bcast_sh)

    plsc.subcore_barrier()
    pltpu.sync_copy(bcast_sh, h1b_v.at[pl.ds(0, 32)])
    c0 = h1b_v[pl.ds(0, 16)]
    c2 = h1b_v[pl.ds(16, 16)]

    s2 = jnp.where(c2 == c0, NB1 + 1, c2)
    base0 = c0 * NB2
    base2 = s2 * NB2
    lim = jnp.uint32(NB2)

    lanes_nb = lanes * jnp.int32(NSLOT * NB2)

    def vec_fn(v):
        key = _keys16(plsc.bitcast(v, jnp.int32))
        g = plsc.bitcast(plsc.bitcast(key, jnp.uint32) >> jnp.uint32(10),
                         jnp.int32)
        dv0 = g - base0
        dv2 = g - base2
        m0 = plsc.bitcast(dv0, jnp.uint32) < lim
        m2 = plsc.bitcast(dv2, jnp.uint32) < lim
        row = jnp.where(m2, dv2 + NB2, dv0)
        return lanes_nb + row, m0 | m2

    _scan_data(x_hbm, d0, d1, semA, semB, wid, vec_fn, hist_v, ones16)

    _fold_stage_reduce(sid, out_hbm, cid * NSLOT * NB2, hist_v, fold_v,
                       stage_sh, rin_v, red_v, NSLOT * NB2)


@functools.partial(
    pl.kernel, mesh=_mesh, compiler_params=_params,
    out_type=[jax.ShapeDtypeStruct((16,), jnp.float32),
              jax.ShapeDtypeStruct((16,), jnp.float32)],
    scratch_types=[
        pltpu.VMEM((NB1,), jnp.int32),
        pltpu.VMEM((NB1,), jnp.int32),
        pltpu.VMEM((NSLOT * NB2,), jnp.int32),
        pltpu.VMEM((NSLOT * NB2,), jnp.int32),
        pltpu.VMEM((16,), jnp.float32),
        pltpu.VMEM((16,), jnp.float32),
    ],
)
def _finish_kernel(h1_hbm, h2_hbm, outp_hbm, outn_hbm, h1a_v, h1b_v, h2a_v,
                   h2b_v, op_v, on_v):
    cid = lax.axis_index("c")
    sid = lax.axis_index("s")
    wid = sid * NC + cid
    lanes = lax.iota(jnp.int32, 16)

    @pl.when(wid == 0)
    def _():
        pltpu.sync_copy(h1_hbm.at[pl.ds(0, NB1)], h1a_v)
        pltpu.sync_copy(h1_hbm.at[pl.ds(NB1, NB1)], h1b_v)
        pltpu.sync_copy(h2_hbm.at[pl.ds(0, NSLOT * NB2)], h2a_v)
        pltpu.sync_copy(h2_hbm.at[pl.ds(NSLOT * NB2, NSLOT * NB2)], h2b_v)

        (c0, c2), belows, tops = _rank_buckets(h1a_v, h1b_v, NB1, _RANKS)
        slot2 = jnp.where(c2 == c0, 0, 1)

        vals = []
        for (slot_base, bucket, below, top, rlo, rhi) in (
                (jnp.zeros((16,), jnp.int32), c0, belows[0], tops[0],
                 _LO_P, _HI_P),
                (slot2 * NB2, c2, belows[1], tops[1], _LO_N, _HI_N)):
            cnt = top - below
            rwb_lo = jnp.minimum(jnp.int32(rlo) - below, cnt - 1)
            rwb_hi = jnp.minimum(jnp.int32(rhi) - below, cnt - 1)

            def cell_body(i, carry, slot_base=slot_base, rwb_lo=rwb_lo,
                          rwb_hi=rwb_hi):
                tot, clo, chi = carry
                idx = slot_base + i * 16 + lanes
                v = (plsc.load_gather(h2a_v, [idx])
                     + plsc.load_gather(h2b_v, [idx]))
                cs = lax.cumsum(v, axis=0) + tot
                clo = clo + plsc.all_reduce_population_count(cs <= rwb_lo)
                chi = chi + plsc.all_reduce_population_count(cs <= rwb_hi)
                return jnp.max(cs), clo, chi

            zeros16 = jnp.zeros((16,), jnp.int32)
            _, cell_lo, cell_hi = lax.fori_loop(
                0, NB2 // 16, cell_body, (jnp.int32(0), zeros16, zeros16))
            for cell in (cell_lo, cell_hi):
                keyj = ((bucket * jnp.int32(1 << 20))
                        | (cell * jnp.int32(1 << 10)) | jnp.int32(512))
                ku = plsc.bitcast(keyj, jnp.uint32)
                neg = (ku & jnp.uint32(0x80000000)) != jnp.uint32(0)
                bits = jnp.where(neg, ku ^ jnp.uint32(0x80000000), ~ku)
                vals.append(plsc.bitcast(bits, jnp.float32))

        qp = vals[0] * jnp.float32(_LW_P) + vals[1] * jnp.float32(_HW_P)
        qn = vals[2] * jnp.float32(_LW_N) + vals[3] * jnp.float32(_HW_N)
        op_v[...] = qp
        on_v[...] = -qn
        pltpu.sync_copy(op_v, outp_hbm)
        pltpu.sync_copy(on_v, outn_hbm)


def kernel(x, scale_p, scale_n):
    x_out = x + jnp.float32(0.0)
    h1 = _hist1_kernel(x)
    h2 = _hist2_kernel(x, h1)
    op16, on16 = _finish_kernel(h1, h2)
    return (x_out, op16[:1], on16[:1])
